# Initial kernel scaffold; baseline (speedup 1.0000x reference)
import jax
import jax.numpy as jnp
from jax.experimental import pallas as pl


def kernel(logits):
    raise NotImplementedError("write your pallas kernel here")


# speedup vs baseline: 5.1426x; 5.1426x over previous
import functools

import jax
import jax.numpy as jnp
from jax import lax
from jax.experimental import pallas as pl
from jax.experimental.pallas import tpu as pltpu
from jax.experimental.pallas import tpu_sc as plsc

N_ROWS = 32768
N_MOTIFS = 64
K = 8

NUM_CORES = 2
NUM_SUBCORES = 16
LANES = 16
NW = NUM_CORES * NUM_SUBCORES
ROWS_PER_W = N_ROWS // NW
CHUNK = 256
N_CHUNKS = ROWS_PER_W // CHUNK
GROUPS = CHUNK // LANES

_MESH = plsc.VectorSubcoreMesh(
    core_axis_name="c", subcore_axis_name="s",
    num_cores=NUM_CORES, num_subcores=NUM_SUBCORES,
)


CHUNK_ELEMS = CHUNK * N_MOTIFS


def _body(logits_hbm, out_hbm, in_v, out_v):
    wid = lax.axis_index("s") * NUM_CORES + lax.axis_index("c")
    elem0 = wid * (ROWS_PER_W * N_MOTIFS)
    lane64 = lax.iota(jnp.int32, LANES) * N_MOTIFS

    def chunk_body(c):
        base = elem0 + c * CHUNK_ELEMS
        pltpu.sync_copy(logits_hbm.at[pl.ds(base, CHUNK_ELEMS)], in_v)

        def group_body(g):
            gbase = g * (LANES * N_MOTIFS) + lane64
            neg_inf = jnp.full((LANES,), -jnp.inf, jnp.float32)

            def p1(j, vs):
                x = plsc.load_gather(in_v, [gbase + j])
                out = []
                for v in vs:
                    t = jnp.maximum(v, x)
                    x = jnp.minimum(v, x)
                    out.append(t)
                return tuple(out)

            vs = lax.fori_loop(0, N_MOTIFS, p1, (neg_inf,) * K, unroll=8)

            mx = vs[0]
            thr = vs[K - 1]
            ngt = jnp.zeros((LANES,), jnp.int32)
            denom = jnp.zeros((LANES,), jnp.float32)
            for v in vs:
                ngt = ngt + jnp.where(v > thr, 1, 0)
                denom = denom + jnp.exp(v - mx)
            budget = 8 - ngt
            scale = 64.0 / denom

            def p2(j, eqcnt):
                idx = gbase + j
                x = plsc.load_gather(in_v, [idx])
                gt = x > thr
                eq = x == thr
                keep = gt | (eq & (eqcnt < budget))
                val = jnp.where(keep, jnp.exp(x - mx) * scale, 0.0)
                plsc.store_scatter(out_v, [idx], val)
                return eqcnt + jnp.where(eq, 1, 0)

            lax.fori_loop(0, N_MOTIFS, p2, jnp.zeros((LANES,), jnp.int32),
                          unroll=8)

        lax.fori_loop(0, GROUPS, lambda g, _: (group_body(g), 0)[1], 0)
        pltpu.sync_copy(out_v, out_hbm.at[pl.ds(base, CHUNK_ELEMS)])

    lax.fori_loop(0, N_CHUNKS, lambda c, _: (chunk_body(c), 0)[1], 0)


@jax.jit
def _router(logits):
    flat = jnp.reshape(logits, (N_ROWS * N_MOTIFS,))
    out = pl.kernel(
        _body,
        out_type=jax.ShapeDtypeStruct((N_ROWS * N_MOTIFS,), jnp.float32),
        mesh=_MESH,
        compiler_params=pltpu.CompilerParams(needs_layout_passes=False),
        scratch_types=[
            pltpu.VMEM((CHUNK_ELEMS,), jnp.float32),
            pltpu.VMEM((CHUNK_ELEMS,), jnp.float32),
        ],
    )(flat)
    return jnp.reshape(out, (N_ROWS, N_MOTIFS))


def kernel(logits):
    return _router(logits)
